# Initial kernel scaffold; baseline (speedup 1.0000x reference)
import jax
import jax.numpy as jnp
from jax.experimental import pallas as pl


def kernel(x_user, x_item, edge_index_item_to_user, edge_index_user_rev_item, W_msg_iu, b_msg_iu, W_msg_ui, b_msg_ui, W_pred1, b_pred1, W_pred2, b_pred2, W_self_user, b_self_user, W_self_item, b_self_item, W_comb_user, b_comb_user, W_comb_item, b_comb_item):
    raise NotImplementedError("write your pallas kernel here")


# speedup vs baseline: 4.2821x; 4.2821x over previous
import functools

import jax
import jax.numpy as jnp
from jax import lax
from jax.experimental import pallas as pl
from jax.experimental.pallas import tpu as pltpu
from jax.experimental.pallas import tpu_sc as plsc

_N = 10000
_E = 160000
_D = 128
_H = 64

_NC = 2
_NS = 16
_DH = _D // _NC

_CHUNK = 128
_CPT = 80
_EPT = _CHUNK * _CPT
_EPAD = _EPT * _NS
_NPAD = 10240
_RPS = _NPAD // _NS
_CW = 16

_TILE_ROWS = 2000


def _sc_segment_sums(tbl_iu, tbl_ui, src_iu, dst_iu, src_ui, dst_ui,
                     zrow, zcnt, ones_h):
    mesh = plsc.VectorSubcoreMesh(core_axis_name="c", subcore_axis_name="s")

    @functools.partial(
        pl.kernel,
        out_type=(
            jax.ShapeDtypeStruct((_NC, _NPAD, _DH), jnp.float32),
            jax.ShapeDtypeStruct((_NPAD, _CW), jnp.float32),
            jax.ShapeDtypeStruct((_NC, _NPAD, _DH), jnp.float32),
            jax.ShapeDtypeStruct((_NPAD, _CW), jnp.float32),
        ),
        mesh=mesh,
        compiler_params=pltpu.CompilerParams(use_tc_tiling_on_sc=False),
        scratch_types=[
            pltpu.VMEM((_CPT, _CHUNK), jnp.int32),
            pltpu.VMEM((_CPT, _CHUNK), jnp.int32),
            pltpu.VMEM((_CHUNK, _DH), jnp.float32),
            pltpu.VMEM((_CHUNK, _CW), jnp.float32),
            pltpu.VMEM_SHARED((_NPAD, _DH), jnp.float32),
            pltpu.VMEM_SHARED((_NPAD, _CW), jnp.float32),
            pltpu.SemaphoreType.DMA,
        ],
    )
    def k(tbl_iu_h, tbl_ui_h, src_iu_h, dst_iu_h, src_ui_h, dst_ui_h,
          zrow_h, zcnt_h, ones_hh,
          out_su, out_cu, out_si, out_ci,
          srcv, dstv, rowsv, onesv, acc, accc, sem):
        cid = lax.axis_index("c")
        sid = lax.axis_index("s")
        r0 = sid * _RPS

        pltpu.sync_copy(ones_hh, onesv)

        def zero_acc():
            pltpu.sync_copy(zrow_h, acc.at[pl.ds(r0, _RPS)])
            pltpu.sync_copy(zcnt_h, accc.at[pl.ds(r0, _RPS)])

        def run_type(src_h, dst_h, table_h, out_s, out_c, cnt_core):
            pltpu.sync_copy(src_h.at[cid, sid], srcv)
            pltpu.sync_copy(dst_h.at[sid], dstv)
            do_cnt = cid == cnt_core

            def body(j, carry):
                pltpu.async_copy(table_h.at[srcv.at[j]], rowsv, sem).wait()
                pltpu.sync_copy(rowsv, acc.at[dstv.at[j]], add=True)

                @pl.when(do_cnt)
                def _():
                    pltpu.sync_copy(onesv, accc.at[dstv.at[j]], add=True)

                return carry

            lax.fori_loop(0, _CPT, body, 0)
            plsc.subcore_barrier()
            pltpu.sync_copy(acc.at[pl.ds(r0, _RPS)],
                            out_s.at[cid, pl.ds(r0, _RPS)])

            @pl.when(do_cnt)
            def _():
                pltpu.sync_copy(accc.at[pl.ds(r0, _RPS)],
                                out_c.at[pl.ds(r0, _RPS)])

        zero_acc()
        plsc.subcore_barrier()
        run_type(src_iu_h, dst_iu_h, tbl_iu_h, out_su, out_cu, 0)
        zero_acc()
        plsc.subcore_barrier()
        run_type(src_ui_h, dst_ui_h, tbl_ui_h, out_si, out_ci, 1)

    return k(tbl_iu, tbl_ui, src_iu, dst_iu, src_ui, dst_ui,
             zrow, zcnt, ones_h)


def _dense_body(xu, xi, su0, su1, cu, si0, si1, ci,
                wmiu, bmiu, wmui, bmui, wp1, bp1, wp2, bp2,
                wsu, bsu, wsi, bsi, wcu, bcu, wci, bci,
                out_u, out_i):
    f32 = jnp.float32
    cuv = cu[:, 0:1]
    gu = jnp.concatenate([su0[...], su1[...]], -1) / jnp.maximum(cuv, 1.0)
    mu = (cuv > 0.0).astype(f32)
    xuv = xu[...]
    h = jnp.maximum(jnp.dot(xuv, wp1[...]) + bp1[...], 0.0)
    pred = jnp.dot(h, wp2[...]) + bp2[...]
    aggstd = jnp.dot(gu, wmiu[...]) + bmiu[...] * mu
    neigh = 0.5 * (aggstd + gu - pred * mu)
    selfu = jnp.dot(xuv, wsu[...]) + bsu[...]
    wcu_t = wcu[...]
    out_u[...] = jnp.maximum(
        jnp.dot(selfu, wcu_t[:_D]) + jnp.dot(neigh, wcu_t[_D:]) + bcu[...], 0.0)
    civ = ci[:, 0:1]
    gi = jnp.concatenate([si0[...], si1[...]], -1) / jnp.maximum(civ, 1.0)
    mi = (civ > 0.0).astype(f32)
    xiv = xi[...]
    aggi = jnp.dot(gi, wmui[...]) + bmui[...] * mi
    selfi = jnp.dot(xiv, wsi[...]) + bsi[...]
    wci_t = wci[...]
    out_i[...] = jnp.maximum(
        jnp.dot(selfi, wci_t[:_D]) + jnp.dot(aggi, wci_t[_D:]) + bci[...], 0.0)


def _dense_combine(xu, xi, su0, su1, cu, si0, si1, ci,
                   wmiu, bmiu, wmui, bmui, wp1, bp1, wp2, bp2,
                   wsu, bsu, wsi, bsi, wcu, bcu, wci, bci):
    grid = _N // _TILE_ROWS
    row = lambda width: pl.BlockSpec((_TILE_ROWS, width), lambda i: (i, 0))
    full = lambda a, b: pl.BlockSpec((a, b), lambda i: (0, 0))
    return pl.pallas_call(
        _dense_body,
        grid=(grid,),
        in_specs=[
            row(_D), row(_D),
            row(_DH), row(_DH), row(_CW),
            row(_DH), row(_DH), row(_CW),
            full(_D, _D), full(1, _D),
            full(_D, _D), full(1, _D),
            full(_D, _H), full(1, _H),
            full(_H, _D), full(1, _D),
            full(_D, _D), full(1, _D),
            full(_D, _D), full(1, _D),
            full(2 * _D, _D), full(1, _D),
            full(2 * _D, _D), full(1, _D),
        ],
        out_specs=[row(_D), row(_D)],
        out_shape=[
            jax.ShapeDtypeStruct((_N, _D), jnp.float32),
            jax.ShapeDtypeStruct((_N, _D), jnp.float32),
        ],
    )(xu, xi, su0, su1, cu, si0, si1, ci,
      wmiu, bmiu, wmui, bmui, wp1, bp1, wp2, bp2,
      wsu, bsu, wsi, bsi, wcu, bcu, wci, bci)


def _prep_edges(ei):
    src = ei[0].astype(jnp.int32)
    dst = ei[1].astype(jnp.int32)
    pad = _EPAD - _E
    src = jnp.concatenate([src, jnp.zeros((pad,), jnp.int32)])
    dst = jnp.concatenate([dst, jnp.full((pad,), _N, jnp.int32)])
    src2 = jnp.stack([src, src + _N]).reshape(_NC, _NS, _CPT, _CHUNK)
    return src2, dst.reshape(_NS, _CPT, _CHUNK)


def _stack_halves(x):
    return jnp.concatenate([x[:, :_DH], x[:, _DH:]], 0)


def kernel(x_user, x_item, edge_index_item_to_user, edge_index_user_rev_item,
           W_msg_iu, b_msg_iu, W_msg_ui, b_msg_ui,
           W_pred1, b_pred1, W_pred2, b_pred2,
           W_self_user, b_self_user, W_self_item, b_self_item,
           W_comb_user, b_comb_user, W_comb_item, b_comb_item):
    x_user = x_user.astype(jnp.float32)
    x_item = x_item.astype(jnp.float32)
    src_iu, dst_iu = _prep_edges(edge_index_item_to_user)
    src_ui, dst_ui = _prep_edges(edge_index_user_rev_item)

    zrow = jnp.zeros((_RPS, _DH), jnp.float32)
    zcnt = jnp.zeros((_RPS, _CW), jnp.float32)
    ones_h = jnp.ones((_CHUNK, _CW), jnp.float32)

    su, cu, si, ci = _sc_segment_sums(
        _stack_halves(x_item), _stack_halves(x_user),
        src_iu, dst_iu, src_ui, dst_ui, zrow, zcnt, ones_h)

    out_u, out_i = _dense_combine(
        x_user, x_item,
        su[0, :_N], su[1, :_N], cu[:_N],
        si[0, :_N], si[1, :_N], ci[:_N],
        W_msg_iu.T, b_msg_iu.reshape(1, _D),
        W_msg_ui.T, b_msg_ui.reshape(1, _D),
        W_pred1.T, b_pred1.reshape(1, _H),
        W_pred2.T, b_pred2.reshape(1, _D),
        W_self_user.T, b_self_user.reshape(1, _D),
        W_self_item.T, b_self_item.reshape(1, _D),
        W_comb_user.T, b_comb_user.reshape(1, _D),
        W_comb_item.T, b_comb_item.reshape(1, _D))
    return (out_u, out_i)


# speedup vs baseline: 8.8071x; 2.0567x over previous
import functools

import jax
import jax.numpy as jnp
from jax import lax
from jax.experimental import pallas as pl
from jax.experimental.pallas import tpu as pltpu
from jax.experimental.pallas import tpu_sc as plsc

_N = 10000
_E = 160000
_D = 128
_H = 64

_NC = 2
_NS = 16
_DH = _D // _NC

_CHUNK = 125
_CPT = 80
_RPS = _N // _NS
_CW = 16

_TILE_ROWS = 2000


def _sc_segment_sums(tbl_iu, tbl_ui, src_iu, dst_iu, src_ui, dst_ui,
                     zrow, zcnt, ones_h):
    mesh = plsc.VectorSubcoreMesh(core_axis_name="c", subcore_axis_name="s")

    @functools.partial(
        pl.kernel,
        out_type=(
            jax.ShapeDtypeStruct((_NC, _N, _DH), jnp.float32),
            jax.ShapeDtypeStruct((_N, _CW), jnp.float32),
            jax.ShapeDtypeStruct((_NC, _N, _DH), jnp.float32),
            jax.ShapeDtypeStruct((_N, _CW), jnp.float32),
        ),
        mesh=mesh,
        compiler_params=pltpu.CompilerParams(use_tc_tiling_on_sc=False),
        scratch_types=[
            pltpu.VMEM((_CPT, _CHUNK), jnp.int32),
            pltpu.VMEM((_CPT, _CHUNK), jnp.int32),
            pltpu.VMEM((_CHUNK, _DH), jnp.float32),
            pltpu.VMEM((_CHUNK, _DH), jnp.float32),
            pltpu.VMEM((_CHUNK, _CW), jnp.float32),
            pltpu.VMEM_SHARED((_N, _DH), jnp.float32),
            pltpu.VMEM_SHARED((_N, _CW), jnp.float32),
            pltpu.SemaphoreType.DMA,
            pltpu.SemaphoreType.DMA,
            pltpu.SemaphoreType.DMA,
            pltpu.SemaphoreType.DMA,
            pltpu.SemaphoreType.DMA,
        ],
    )
    def k(tbl_iu_h, tbl_ui_h, src_iu_h, dst_iu_h, src_ui_h, dst_ui_h,
          zrow_h, zcnt_h, ones_hh,
          out_su, out_cu, out_si, out_ci,
          srcv, dstv, rowsa, rowsb, onesv, acc, accc,
          sga, sgb, ssa, ssb, sc):
        cid = lax.axis_index("c")
        sid = lax.axis_index("s")
        r0 = sid * _RPS

        pltpu.sync_copy(ones_hh, onesv)

        def zero_acc():
            pltpu.sync_copy(zrow_h, acc.at[pl.ds(r0, _RPS)])
            pltpu.sync_copy(zcnt_h, accc.at[pl.ds(r0, _RPS)])

        def run_type(src_h, dst_h, table_h, out_s, out_c, cnt_core):
            pltpu.sync_copy(src_h.at[cid, sid], srcv)
            pltpu.sync_copy(dst_h.at[sid], dstv)
            do_cnt = cid == cnt_core

            def fire_gather(j, buf, sem):
                pltpu.async_copy(table_h.at[srcv.at[j]], buf, sem)

            def wait_gather(j, buf, sem):
                pltpu.make_async_copy(table_h.at[srcv.at[j]], buf, sem).wait()

            def fire_scatter(j, buf, sem):
                pltpu.async_copy(buf, acc.at[dstv.at[j]], sem, add=True)

                @pl.when(do_cnt)
                def _():
                    pltpu.async_copy(onesv, accc.at[dstv.at[j]], sc, add=True)

            def wait_scatter(j, buf, sem):
                pltpu.make_async_copy(buf, acc.at[dstv.at[j]], sem).wait()

                @pl.when(do_cnt)
                def _():
                    pltpu.make_async_copy(onesv, accc.at[dstv.at[j]], sc).wait()

            fire_gather(0, rowsa, sga)

            def body(i, carry):
                j = 2 * i
                fire_gather(j + 1, rowsb, sgb)
                wait_gather(j, rowsa, sga)
                fire_scatter(j, rowsa, ssa)
                wait_scatter(j, rowsa, ssa)

                @pl.when(j + 2 < _CPT)
                def _():
                    fire_gather(j + 2, rowsa, sga)

                wait_gather(j + 1, rowsb, sgb)
                fire_scatter(j + 1, rowsb, ssb)
                wait_scatter(j + 1, rowsb, ssb)
                return carry

            lax.fori_loop(0, _CPT // 2, body, 0)
            plsc.subcore_barrier()
            pltpu.sync_copy(acc.at[pl.ds(r0, _RPS)],
                            out_s.at[cid, pl.ds(r0, _RPS)])

            @pl.when(do_cnt)
            def _():
                pltpu.sync_copy(accc.at[pl.ds(r0, _RPS)],
                                out_c.at[pl.ds(r0, _RPS)])

        zero_acc()
        plsc.subcore_barrier()
        run_type(src_iu_h, dst_iu_h, tbl_iu_h, out_su, out_cu, 0)
        zero_acc()
        plsc.subcore_barrier()
        run_type(src_ui_h, dst_ui_h, tbl_ui_h, out_si, out_ci, 1)

    return k(tbl_iu, tbl_ui, src_iu, dst_iu, src_ui, dst_ui,
             zrow, zcnt, ones_h)


def _dense_body(xu, xi, su, cu, si, ci,
                wmiu, bmiu, wmui, bmui, wp1, bp1, wp2, bp2,
                wsu, bsu, wsi, bsi, wcu, bcu, wci, bci,
                out_u, out_i):
    f32 = jnp.float32
    cuv = cu[:, 0:1]
    gu = jnp.concatenate([su[0], su[1]], -1) / jnp.maximum(cuv, 1.0)
    mu = (cuv > 0.0).astype(f32)
    xuv = xu[...]
    h = jnp.maximum(jnp.dot(xuv, wp1[...]) + bp1[...], 0.0)
    pred = jnp.dot(h, wp2[...]) + bp2[...]
    aggstd = jnp.dot(gu, wmiu[...]) + bmiu[...] * mu
    neigh = 0.5 * (aggstd + gu - pred * mu)
    selfu = jnp.dot(xuv, wsu[...]) + bsu[...]
    wcu_t = wcu[...]
    out_u[...] = jnp.maximum(
        jnp.dot(selfu, wcu_t[:_D]) + jnp.dot(neigh, wcu_t[_D:]) + bcu[...], 0.0)
    civ = ci[:, 0:1]
    gi = jnp.concatenate([si[0], si[1]], -1) / jnp.maximum(civ, 1.0)
    mi = (civ > 0.0).astype(f32)
    xiv = xi[...]
    aggi = jnp.dot(gi, wmui[...]) + bmui[...] * mi
    selfi = jnp.dot(xiv, wsi[...]) + bsi[...]
    wci_t = wci[...]
    out_i[...] = jnp.maximum(
        jnp.dot(selfi, wci_t[:_D]) + jnp.dot(aggi, wci_t[_D:]) + bci[...], 0.0)


def _dense_combine(xu, xi, su, cu, si, ci,
                   wmiu, bmiu, wmui, bmui, wp1, bp1, wp2, bp2,
                   wsu, bsu, wsi, bsi, wcu, bcu, wci, bci):
    grid = _N // _TILE_ROWS
    row = lambda width: pl.BlockSpec((_TILE_ROWS, width), lambda i: (i, 0))
    half = pl.BlockSpec((_NC, _TILE_ROWS, _DH), lambda i: (0, i, 0))
    full = lambda a, b: pl.BlockSpec((a, b), lambda i: (0, 0))
    return pl.pallas_call(
        _dense_body,
        grid=(grid,),
        in_specs=[
            row(_D), row(_D),
            half, row(_CW),
            half, row(_CW),
            full(_D, _D), full(1, _D),
            full(_D, _D), full(1, _D),
            full(_D, _H), full(1, _H),
            full(_H, _D), full(1, _D),
            full(_D, _D), full(1, _D),
            full(_D, _D), full(1, _D),
            full(2 * _D, _D), full(1, _D),
            full(2 * _D, _D), full(1, _D),
        ],
        out_specs=[row(_D), row(_D)],
        out_shape=[
            jax.ShapeDtypeStruct((_N, _D), jnp.float32),
            jax.ShapeDtypeStruct((_N, _D), jnp.float32),
        ],
    )(xu, xi, su, cu, si, ci,
      wmiu, bmiu, wmui, bmui, wp1, bp1, wp2, bp2,
      wsu, bsu, wsi, bsi, wcu, bcu, wci, bci)


def _prep_edges(ei):
    src = ei[0].astype(jnp.int32)
    dst = ei[1].astype(jnp.int32)
    src2 = jnp.stack([src, src + _N]).reshape(_NC, _NS, _CPT, _CHUNK)
    return src2, dst.reshape(_NS, _CPT, _CHUNK)


def _stack_halves(x):
    return jnp.concatenate([x[:, :_DH], x[:, _DH:]], 0)


def kernel(x_user, x_item, edge_index_item_to_user, edge_index_user_rev_item,
           W_msg_iu, b_msg_iu, W_msg_ui, b_msg_ui,
           W_pred1, b_pred1, W_pred2, b_pred2,
           W_self_user, b_self_user, W_self_item, b_self_item,
           W_comb_user, b_comb_user, W_comb_item, b_comb_item):
    x_user = x_user.astype(jnp.float32)
    x_item = x_item.astype(jnp.float32)
    src_iu, dst_iu = _prep_edges(edge_index_item_to_user)
    src_ui, dst_ui = _prep_edges(edge_index_user_rev_item)

    zrow = jnp.zeros((_RPS, _DH), jnp.float32)
    zcnt = jnp.zeros((_RPS, _CW), jnp.float32)
    ones_h = jnp.ones((_CHUNK, _CW), jnp.float32)

    su, cu, si, ci = _sc_segment_sums(
        _stack_halves(x_item), _stack_halves(x_user),
        src_iu, dst_iu, src_ui, dst_ui, zrow, zcnt, ones_h)

    out_u, out_i = _dense_combine(
        x_user, x_item, su, cu, si, ci,
        W_msg_iu.T, b_msg_iu.reshape(1, _D),
        W_msg_ui.T, b_msg_ui.reshape(1, _D),
        W_pred1.T, b_pred1.reshape(1, _H),
        W_pred2.T, b_pred2.reshape(1, _D),
        W_self_user.T, b_self_user.reshape(1, _D),
        W_self_item.T, b_self_item.reshape(1, _D),
        W_comb_user.T, b_comb_user.reshape(1, _D),
        W_comb_item.T, b_comb_item.reshape(1, _D))
    return (out_u, out_i)


# speedup vs baseline: 9.6285x; 1.0933x over previous
import functools

import jax
import jax.numpy as jnp
from jax import lax
from jax.experimental import pallas as pl
from jax.experimental.pallas import tpu as pltpu
from jax.experimental.pallas import tpu_sc as plsc

_N = 10000
_E = 160000
_D = 128
_H = 64

_NC = 2
_NS = 16
_NW = _NC * _NS

_CHUNK = 50
_CPT = 100
_RPS = _N // _NS
_CW = 16

_TILE_ROWS = 2000


def _sc_segment_sums(x_item, x_user, src_iu, dst_iu, src_ui, dst_ui,
                     zrow, zcnt, ones_h):
    mesh = plsc.VectorSubcoreMesh(core_axis_name="c", subcore_axis_name="s")

    @functools.partial(
        pl.kernel,
        out_type=(
            jax.ShapeDtypeStruct((_NC, _N, _D), jnp.float32),
            jax.ShapeDtypeStruct((_NC, _N, _CW), jnp.float32),
            jax.ShapeDtypeStruct((_NC, _N, _D), jnp.float32),
            jax.ShapeDtypeStruct((_NC, _N, _CW), jnp.float32),
        ),
        mesh=mesh,
        compiler_params=pltpu.CompilerParams(use_tc_tiling_on_sc=False),
        scratch_types=[
            pltpu.VMEM((_CPT, _CHUNK), jnp.int32),
            pltpu.VMEM((_CPT, _CHUNK), jnp.int32),
            pltpu.VMEM((_CHUNK, _D), jnp.float32),
            pltpu.VMEM((_CHUNK, _D), jnp.float32),
            pltpu.VMEM((_CHUNK, _CW), jnp.float32),
            pltpu.VMEM_SHARED((_N, _D), jnp.float32),
            pltpu.VMEM_SHARED((_N, _CW), jnp.float32),
            pltpu.SemaphoreType.DMA,
            pltpu.SemaphoreType.DMA,
            pltpu.SemaphoreType.DMA,
            pltpu.SemaphoreType.DMA,
            pltpu.SemaphoreType.DMA,
        ],
    )
    def k(x_item_h, x_user_h, src_iu_h, dst_iu_h, src_ui_h, dst_ui_h,
          zrow_h, zcnt_h, ones_hh,
          out_su, out_cu, out_si, out_ci,
          srcv, dstv, rowsa, rowsb, onesv, acc, accc,
          sga, sgb, ssa, ssb, sc):
        cid = lax.axis_index("c")
        sid = lax.axis_index("s")
        wid = sid * _NC + cid
        r0 = sid * _RPS

        pltpu.sync_copy(ones_hh, onesv)

        def zero_acc():
            pltpu.sync_copy(zrow_h, acc.at[pl.ds(r0, _RPS)])
            pltpu.sync_copy(zcnt_h, accc.at[pl.ds(r0, _RPS)])

        def run_type(src_h, dst_h, table_h, out_s, out_c):
            pltpu.sync_copy(src_h.at[wid], srcv)
            pltpu.sync_copy(dst_h.at[wid], dstv)

            def fire_gather(j, buf, sem):
                pltpu.async_copy(table_h.at[srcv.at[j]], buf, sem)

            def wait_gather(j, buf, sem):
                pltpu.make_async_copy(table_h.at[srcv.at[j]], buf, sem).wait()

            def fire_scatter(j, buf, sem):
                pltpu.async_copy(buf, acc.at[dstv.at[j]], sem, add=True)
                pltpu.async_copy(onesv, accc.at[dstv.at[j]], sc, add=True)

            def wait_scatter(j, buf, sem):
                pltpu.make_async_copy(buf, acc.at[dstv.at[j]], sem).wait()
                pltpu.make_async_copy(onesv, accc.at[dstv.at[j]], sc).wait()

            fire_gather(0, rowsa, sga)

            def body(i, carry):
                j = 2 * i
                fire_gather(j + 1, rowsb, sgb)
                wait_gather(j, rowsa, sga)
                fire_scatter(j, rowsa, ssa)
                wait_scatter(j, rowsa, ssa)

                @pl.when(j + 2 < _CPT)
                def _():
                    fire_gather(j + 2, rowsa, sga)

                wait_gather(j + 1, rowsb, sgb)
                fire_scatter(j + 1, rowsb, ssb)
                wait_scatter(j + 1, rowsb, ssb)
                return carry

            lax.fori_loop(0, _CPT // 2, body, 0)
            plsc.subcore_barrier()
            pltpu.sync_copy(acc.at[pl.ds(r0, _RPS)],
                            out_s.at[cid, pl.ds(r0, _RPS)])
            pltpu.sync_copy(accc.at[pl.ds(r0, _RPS)],
                            out_c.at[cid, pl.ds(r0, _RPS)])

        zero_acc()
        plsc.subcore_barrier()
        run_type(src_iu_h, dst_iu_h, x_item_h, out_su, out_cu)
        zero_acc()
        plsc.subcore_barrier()
        run_type(src_ui_h, dst_ui_h, x_user_h, out_si, out_ci)

    return k(x_item, x_user, src_iu, dst_iu, src_ui, dst_ui,
             zrow, zcnt, ones_h)


def _dense_body(xu, xi, su, cu, si, ci,
                wmiu, bmiu, wmui, bmui, wp1, bp1, wp2, bp2,
                wsu, bsu, wsi, bsi, wcu, bcu, wci, bci,
                out_u, out_i):
    f32 = jnp.float32
    cuv = cu[0, :, 0:1] + cu[1, :, 0:1]
    gu = (su[0] + su[1]) / jnp.maximum(cuv, 1.0)
    mu = (cuv > 0.0).astype(f32)
    xuv = xu[...]
    h = jnp.maximum(jnp.dot(xuv, wp1[...]) + bp1[...], 0.0)
    pred = jnp.dot(h, wp2[...]) + bp2[...]
    aggstd = jnp.dot(gu, wmiu[...]) + bmiu[...] * mu
    neigh = 0.5 * (aggstd + gu - pred * mu)
    selfu = jnp.dot(xuv, wsu[...]) + bsu[...]
    wcu_t = wcu[...]
    out_u[...] = jnp.maximum(
        jnp.dot(selfu, wcu_t[:_D]) + jnp.dot(neigh, wcu_t[_D:]) + bcu[...], 0.0)
    civ = ci[0, :, 0:1] + ci[1, :, 0:1]
    gi = (si[0] + si[1]) / jnp.maximum(civ, 1.0)
    mi = (civ > 0.0).astype(f32)
    xiv = xi[...]
    aggi = jnp.dot(gi, wmui[...]) + bmui[...] * mi
    selfi = jnp.dot(xiv, wsi[...]) + bsi[...]
    wci_t = wci[...]
    out_i[...] = jnp.maximum(
        jnp.dot(selfi, wci_t[:_D]) + jnp.dot(aggi, wci_t[_D:]) + bci[...], 0.0)


def _dense_combine(xu, xi, su, cu, si, ci,
                   wmiu, bmiu, wmui, bmui, wp1, bp1, wp2, bp2,
                   wsu, bsu, wsi, bsi, wcu, bcu, wci, bci):
    grid = _N // _TILE_ROWS
    row = lambda width: pl.BlockSpec((_TILE_ROWS, width), lambda i: (i, 0))
    pair = lambda width: pl.BlockSpec((_NC, _TILE_ROWS, width), lambda i: (0, i, 0))
    full = lambda a, b: pl.BlockSpec((a, b), lambda i: (0, 0))
    return pl.pallas_call(
        _dense_body,
        grid=(grid,),
        in_specs=[
            row(_D), row(_D),
            pair(_D), pair(_CW),
            pair(_D), pair(_CW),
            full(_D, _D), full(1, _D),
            full(_D, _D), full(1, _D),
            full(_D, _H), full(1, _H),
            full(_H, _D), full(1, _D),
            full(_D, _D), full(1, _D),
            full(_D, _D), full(1, _D),
            full(2 * _D, _D), full(1, _D),
            full(2 * _D, _D), full(1, _D),
        ],
        out_specs=[row(_D), row(_D)],
        out_shape=[
            jax.ShapeDtypeStruct((_N, _D), jnp.float32),
            jax.ShapeDtypeStruct((_N, _D), jnp.float32),
        ],
    )(xu, xi, su, cu, si, ci,
      wmiu, bmiu, wmui, bmui, wp1, bp1, wp2, bp2,
      wsu, bsu, wsi, bsi, wcu, bcu, wci, bci)


def _prep_edges(ei):
    src = ei[0].astype(jnp.int32).reshape(_NW, _CPT, _CHUNK)
    dst = ei[1].astype(jnp.int32).reshape(_NW, _CPT, _CHUNK)
    return src, dst


def kernel(x_user, x_item, edge_index_item_to_user, edge_index_user_rev_item,
           W_msg_iu, b_msg_iu, W_msg_ui, b_msg_ui,
           W_pred1, b_pred1, W_pred2, b_pred2,
           W_self_user, b_self_user, W_self_item, b_self_item,
           W_comb_user, b_comb_user, W_comb_item, b_comb_item):
    x_user = x_user.astype(jnp.float32)
    x_item = x_item.astype(jnp.float32)
    src_iu, dst_iu = _prep_edges(edge_index_item_to_user)
    src_ui, dst_ui = _prep_edges(edge_index_user_rev_item)

    zrow = jnp.zeros((_RPS, _D), jnp.float32)
    zcnt = jnp.zeros((_RPS, _CW), jnp.float32)
    ones_h = jnp.ones((_CHUNK, _CW), jnp.float32)

    su, cu, si, ci = _sc_segment_sums(
        x_item, x_user, src_iu, dst_iu, src_ui, dst_ui, zrow, zcnt, ones_h)

    out_u, out_i = _dense_combine(
        x_user, x_item, su, cu, si, ci,
        W_msg_iu.T, b_msg_iu.reshape(1, _D),
        W_msg_ui.T, b_msg_ui.reshape(1, _D),
        W_pred1.T, b_pred1.reshape(1, _H),
        W_pred2.T, b_pred2.reshape(1, _D),
        W_self_user.T, b_self_user.reshape(1, _D),
        W_self_item.T, b_self_item.reshape(1, _D),
        W_comb_user.T, b_comb_user.reshape(1, _D),
        W_comb_item.T, b_comb_item.reshape(1, _D))
    return (out_u, out_i)
